# baseline (speedup 1.0000x reference)
import functools
import math

import jax
import jax.numpy as jnp
from jax import lax
from jax.experimental import pallas as pl
from jax.experimental.pallas import tpu as pltpu
from jax.experimental.pallas import tpu_sc as plsc

N = 5000
NPAD = 5120
LIMIT = 20000
OUT_PAD = LIMIT + 16
NW = 32
NC = 2
CAP = 20480
CHUNK = 512
SLOT = 624
GROWS = 6
UNROLL = 8


def _row_bounds():
    bs = [0]
    for w in range(1, NW):
        r = round(N * (1.0 - math.sqrt(1.0 - w / NW)))
        bs.append(max(bs[-1], min(r, N)))
    bs.append(N)
    return bs


_BOUNDS = _row_bounds()


def _wid():
    return lax.axis_index("s") * NC + lax.axis_index("c")


def _vscalar(vec, iota):
    return jnp.sum(jnp.where(iota == 0, vec, 0))


def _phase1_body(xmin_h, ymin_h, xmax_h, ymax_h, bounds_h,
                 counts_h, ei_h, ej_h, ex_h, ey_h,
                 xmin_v, ymin_v, xmax_v, ymax_v, bounds_v,
                 bi, bj, bx, by, stage, sem):
    w = _wid()
    iota = lax.iota(jnp.int32, 16)

    pltpu.sync_copy(xmin_h, xmin_v)
    pltpu.sync_copy(ymin_h, ymin_v)
    pltpu.sync_copy(xmax_h, xmax_v)
    pltpu.sync_copy(ymax_h, ymax_v)
    pltpu.sync_copy(bounds_h, bounds_v)

    bvec_lo = plsc.load_gather(bounds_v, [jnp.full((16,), w, jnp.int32)])
    bvec_hi = plsc.load_gather(bounds_v, [jnp.full((16,), w + 1, jnp.int32)])
    rs = _vscalar(bvec_lo, iota)
    re = _vscalar(bvec_hi, iota)

    def row_body(i, off_v):
        isplat = jnp.full((16,), i, jnp.int32)
        rxmin = plsc.load_gather(xmin_v, [isplat])
        rymin = plsc.load_gather(ymin_v, [isplat])
        rxmax = plsc.load_gather(xmax_v, [isplat])
        rymax = plsc.load_gather(ymax_v, [isplat])
        jc0 = (i + 1) // (16 * UNROLL)

        def chunk_body(jcq, off_v):
            base0 = jcq * (16 * UNROLL)
            offs = off_v
            for u in range(UNROLL):
                base = base0 + u * 16
                cxmin = xmin_v[pl.ds(base, 16)]
                cymin = ymin_v[pl.ds(base, 16)]
                cxmax = xmax_v[pl.ds(base, 16)]
                cymax = ymax_v[pl.ds(base, 16)]
                ox = jnp.minimum(rxmax, cxmax) - jnp.maximum(rxmin, cxmin)
                oy = jnp.minimum(rymax, cymax) - jnp.maximum(rymin, cymin)
                jv = base + iota
                m = (jnp.minimum(ox, oy) > 0.0) & (jv > i)
                pos = plsc.cumsum(m.astype(jnp.int32))
                offc = jnp.minimum(offs, CAP - 16)
                idx = offc + pos - 1
                plsc.store_scatter(bi, [idx], isplat, mask=m)
                plsc.store_scatter(bj, [idx], jv, mask=m)
                plsc.store_scatter(bx, [idx], ox, mask=m)
                plsc.store_scatter(by, [idx], oy, mask=m)
                offs = offs + plsc.all_reduce_population_count(m)
            return offs

        return lax.fori_loop(jc0, NPAD // (16 * UNROLL), chunk_body, off_v)

    off_v = lax.fori_loop(rs, re, row_body, jnp.zeros((16,), jnp.int32))
    count = jnp.max(off_v)

    stage[...] = jnp.full((16,), count, jnp.int32)
    pltpu.sync_copy(stage, counts_h.at[pl.ds(w * 16, 16)])

    nch = (jnp.minimum(count, CAP) + CHUNK - 1) // CHUNK

    def spill(k, _):
        src = pl.ds(k * CHUNK, CHUNK)
        dst = pl.ds(w * CAP + k * CHUNK, CHUNK)
        pltpu.sync_copy(bi.at[src], ei_h.at[dst])
        pltpu.sync_copy(bj.at[src], ej_h.at[dst])
        pltpu.sync_copy(bx.at[src], ex_h.at[dst])
        pltpu.sync_copy(by.at[src], ey_h.at[dst])
        return 0

    lax.fori_loop(0, nch, spill, 0)

    @pl.when(w == 0)
    def _():
        stage[...] = jnp.full((16,), -1, jnp.int32)
        pltpu.sync_copy(stage, ei_h.at[pl.ds(NW * CAP, 16)])
        pltpu.sync_copy(stage, ej_h.at[pl.ds(NW * CAP, 16)])
        bx[pl.ds(0, 16)] = jnp.zeros((16,), jnp.float32)
        pltpu.sync_copy(bx.at[pl.ds(0, 16)], ex_h.at[pl.ds(NW * CAP, 16)])
        pltpu.sync_copy(bx.at[pl.ds(0, 16)], ey_h.at[pl.ds(NW * CAP, 16)])


_PHASE1_KWARGS = dict(
    out_type=[
        jax.ShapeDtypeStruct((NW * 16,), jnp.int32),
        jax.ShapeDtypeStruct((NW * CAP + 16,), jnp.int32),
        jax.ShapeDtypeStruct((NW * CAP + 16,), jnp.int32),
        jax.ShapeDtypeStruct((NW * CAP + 16,), jnp.float32),
        jax.ShapeDtypeStruct((NW * CAP + 16,), jnp.float32),
    ],
    scratch_types=[
        pltpu.VMEM((NPAD,), jnp.float32),
        pltpu.VMEM((NPAD,), jnp.float32),
        pltpu.VMEM((NPAD,), jnp.float32),
        pltpu.VMEM((NPAD,), jnp.float32),
        pltpu.VMEM((48,), jnp.int32),
        pltpu.VMEM((CAP,), jnp.int32),
        pltpu.VMEM((CAP,), jnp.int32),
        pltpu.VMEM((CAP,), jnp.float32),
        pltpu.VMEM((CAP,), jnp.float32),
        pltpu.VMEM((16,), jnp.int32),
        pltpu.SemaphoreType.DMA,
    ],
)


def _phase2_body(counts_h, ei_h, ej_h, ex_h, ey_h, lim_h,
                 oi_h, oj_h, opx_h, opy_h,
                 cnt_v, lim_v, obuf, gidx, gbi, gbj, gbx, gby, sem):
    w = _wid()
    iota = lax.iota(jnp.int32, 16)

    pltpu.sync_copy(counts_h, cnt_v)
    pltpu.sync_copy(lim_h, lim_v)

    c0 = plsc.load_gather(cnt_v, [iota * 16])
    c1 = plsc.load_gather(cnt_v, [256 + iota * 16])
    lim = _vscalar(lim_v[...], iota)
    s0 = jnp.sum(c0)
    e0 = plsc.cumsum(c0) - c0
    e1 = plsc.cumsum(c1) - c1 + s0
    k_total = s0 + jnp.sum(c1)
    mink = jnp.minimum(k_total, lim)
    obuf[pl.ds(0, 16)] = e0
    obuf[pl.ds(16, 16)] = e1

    rs = w * SLOT

    def sixteen(t, _):
        g = rs + t * 16 + iota

        def owner(s, acc):
            o_s = plsc.load_gather(obuf, [jnp.full((16,), s, jnp.int32)])
            return acc + (g >= o_s).astype(jnp.int32)

        v = lax.fori_loop(0, NW, owner, jnp.zeros((16,), jnp.int32)) - 1
        o_v = plsc.load_gather(obuf, [v])
        addr = v * CAP + g - o_v
        addr = jnp.where(g < mink, addr, NW * CAP + iota)
        gidx[pl.ds(t * 16, 16)] = addr
        return 0

    lax.fori_loop(0, GROWS * 8, sixteen, 0)

    dmas = []
    for r in range(GROWS):
        sl = pl.ds(r * 128, 128)
        idx = gidx.at[sl]
        dmas.append(pltpu.async_copy(ei_h.at[idx], gbi.at[sl], sem))
        dmas.append(pltpu.async_copy(ej_h.at[idx], gbj.at[sl], sem))
        dmas.append(pltpu.async_copy(ex_h.at[idx], gbx.at[sl], sem))
        dmas.append(pltpu.async_copy(ey_h.at[idx], gby.at[sl], sem))
    for d in dmas:
        d.wait()

    dmas = [
        pltpu.async_copy(gbi.at[pl.ds(0, SLOT)], oi_h.at[pl.ds(rs, SLOT)], sem),
        pltpu.async_copy(gbj.at[pl.ds(0, SLOT)], oj_h.at[pl.ds(rs, SLOT)], sem),
        pltpu.async_copy(gbx.at[pl.ds(0, SLOT)], opx_h.at[pl.ds(rs, SLOT)], sem),
        pltpu.async_copy(gby.at[pl.ds(0, SLOT)], opy_h.at[pl.ds(rs, SLOT)], sem),
    ]
    for d in dmas:
        d.wait()

    @pl.when(w == NW - 1)
    def _():
        src = pl.ds(SLOT, LIMIT - NW * SLOT)
        dst = pl.ds(rs + SLOT, LIMIT - NW * SLOT)
        d1 = pltpu.async_copy(gbi.at[src], oi_h.at[dst], sem)
        d2 = pltpu.async_copy(gbj.at[src], oj_h.at[dst], sem)
        d3 = pltpu.async_copy(gbx.at[src], opx_h.at[dst], sem)
        d4 = pltpu.async_copy(gby.at[src], opy_h.at[dst], sem)
        d1.wait()
        d2.wait()
        d3.wait()
        d4.wait()


_PHASE2_KWARGS = dict(
    out_type=[
        jax.ShapeDtypeStruct((LIMIT,), jnp.int32),
        jax.ShapeDtypeStruct((LIMIT,), jnp.int32),
        jax.ShapeDtypeStruct((LIMIT,), jnp.float32),
        jax.ShapeDtypeStruct((LIMIT,), jnp.float32),
    ],
    scratch_types=[
        pltpu.VMEM((NW * 16,), jnp.int32),
        pltpu.VMEM((16,), jnp.int32),
        pltpu.VMEM((NW,), jnp.int32),
        pltpu.VMEM((GROWS * 128,), jnp.int32),
        pltpu.VMEM((GROWS * 128,), jnp.int32),
        pltpu.VMEM((GROWS * 128,), jnp.int32),
        pltpu.VMEM((GROWS * 128,), jnp.float32),
        pltpu.VMEM((GROWS * 128,), jnp.float32),
        pltpu.SemaphoreType.DMA,
    ],
)


@functools.cache
def _build_phases():
    mesh = plsc.VectorSubcoreMesh(core_axis_name="c", subcore_axis_name="s",
                                  num_cores=NC, num_subcores=NW // NC)
    params = pltpu.CompilerParams(needs_layout_passes=False,
                                  disable_bounds_checks=True)
    phase1 = pl.kernel(_phase1_body, mesh=mesh, compiler_params=params,
                       **_PHASE1_KWARGS)
    phase2 = pl.kernel(_phase2_body, mesh=mesh, compiler_params=params,
                       **_PHASE2_KWARGS)
    return phase1, phase2


def kernel(boxes, limit):
    boxes = boxes.astype(jnp.float32)
    padlo = jnp.full((NPAD - N,), 2e9, jnp.float32)
    padhi = jnp.full((NPAD - N,), -2e9, jnp.float32)
    xmin = jnp.concatenate([boxes[:, 0], padlo])
    ymin = jnp.concatenate([boxes[:, 1], padlo])
    xmax = jnp.concatenate([boxes[:, 2], padhi])
    ymax = jnp.concatenate([boxes[:, 3], padhi])
    bounds = jnp.asarray(_BOUNDS + [N] * (48 - len(_BOUNDS)), jnp.int32)
    lim = jnp.minimum(jnp.asarray(limit, jnp.int32), LIMIT)
    lim16 = jnp.full((16,), lim, jnp.int32)

    phase1, phase2 = _build_phases()
    counts, ei, ej, ex, ey = phase1(xmin, ymin, xmax, ymax, bounds)
    oi, oj, opx, opy = phase2(counts, ei, ej, ex, ey, lim16)

    pairs = jnp.stack([oi, oj], axis=-1)
    pen = jnp.stack([opx, opy], axis=-1)
    return pairs, pen


# speedup vs baseline: 3.1899x; 3.1899x over previous
import functools
import math

import jax
import jax.numpy as jnp
from jax import lax
from jax.experimental import pallas as pl
from jax.experimental.pallas import tpu as pltpu
from jax.experimental.pallas import tpu_sc as plsc

N = 5000
NPAD = 5120
LIMIT = 20000
OUT_PAD = LIMIT + 16
NW = 32
NC = 2
CAP = 20480
CHUNK = 512
SLOT = 624
GROWS = 6
UNROLL = 8


def _row_bounds():
    bs = [0]
    for w in range(1, NW):
        r = round(N * (1.0 - math.sqrt(1.0 - w / NW)))
        bs.append(max(bs[-1], min(r, N)))
    bs.append(N)
    return bs


_BOUNDS = _row_bounds()


def _wid():
    return lax.axis_index("s") * NC + lax.axis_index("c")


def _vscalar(vec, iota):
    return jnp.sum(jnp.where(iota == 0, vec, 0))


def _phase1_body(xmin_h, ymin_h, xmax_h, ymax_h, bounds_h,
                 counts_h, eg_h,
                 xmin_v, ymin_v, xmax_v, ymax_v, bounds_v,
                 bg, stage, sem):
    w = _wid()
    iota = lax.iota(jnp.int32, 16)

    pltpu.sync_copy(xmin_h, xmin_v)
    pltpu.sync_copy(ymin_h, ymin_v)
    pltpu.sync_copy(xmax_h, xmax_v)
    pltpu.sync_copy(ymax_h, ymax_v)
    pltpu.sync_copy(bounds_h, bounds_v)

    bvec_lo = plsc.load_gather(bounds_v, [jnp.full((16,), w, jnp.int32)])
    bvec_hi = plsc.load_gather(bounds_v, [jnp.full((16,), w + 1, jnp.int32)])
    rs = _vscalar(bvec_lo, iota)
    re = _vscalar(bvec_hi, iota)

    def row_body(i, off_v):
        isplat = jnp.full((16,), i, jnp.int32)
        ishift = isplat << 13
        rxmin = plsc.load_gather(xmin_v, [isplat])
        rymin = plsc.load_gather(ymin_v, [isplat])
        rxmax = plsc.load_gather(xmax_v, [isplat])
        rymax = plsc.load_gather(ymax_v, [isplat])
        lo = ((i + 1) // (16 * UNROLL)) * UNROLL

        def chunk_body(jc, offs):
            base = jc * 16
            cxmin = xmin_v[pl.ds(base, 16)]
            cymin = ymin_v[pl.ds(base, 16)]
            cxmax = xmax_v[pl.ds(base, 16)]
            cymax = ymax_v[pl.ds(base, 16)]
            ox = jnp.minimum(rxmax, cxmax) - jnp.maximum(rxmin, cxmin)
            oy = jnp.minimum(rymax, cymax) - jnp.maximum(rymin, cymin)
            jv = base + iota
            m = (jnp.minimum(ox, oy) > 0.0) & (jv > i)
            pos = plsc.cumsum(m.astype(jnp.int32))
            offc = jnp.minimum(offs, CAP - 16)
            plsc.store_scatter(bg, [offc + pos - 1], ishift | jv, mask=m)
            return offs + plsc.all_reduce_population_count(m)

        return plsc.parallel_loop(lo, NPAD // 16, unroll=UNROLL,
                                  carry=off_v)(chunk_body)

    off_v = lax.fori_loop(rs, re, row_body, jnp.zeros((16,), jnp.int32))
    count = jnp.max(off_v)

    stage[...] = jnp.full((16,), count, jnp.int32)
    pltpu.sync_copy(stage, counts_h.at[pl.ds(w * 16, 16)])

    nch = (jnp.minimum(count, CAP) + CHUNK - 1) // CHUNK

    def spill(k, _):
        pltpu.sync_copy(bg.at[pl.ds(k * CHUNK, CHUNK)],
                        eg_h.at[pl.ds(w * CAP + k * CHUNK, CHUNK)])
        return 0

    lax.fori_loop(0, nch, spill, 0)

    @pl.when(w == 0)
    def _():
        stage[...] = jnp.full((16,), -1, jnp.int32)
        pltpu.sync_copy(stage, eg_h.at[pl.ds(NW * CAP, 16)])


_PHASE1_KWARGS = dict(
    out_type=[
        jax.ShapeDtypeStruct((NW * 16,), jnp.int32),
        jax.ShapeDtypeStruct((NW * CAP + 16,), jnp.int32),
    ],
    scratch_types=[
        pltpu.VMEM((NPAD,), jnp.float32),
        pltpu.VMEM((NPAD,), jnp.float32),
        pltpu.VMEM((NPAD,), jnp.float32),
        pltpu.VMEM((NPAD,), jnp.float32),
        pltpu.VMEM((48,), jnp.int32),
        pltpu.VMEM((CAP,), jnp.int32),
        pltpu.VMEM((16,), jnp.int32),
        pltpu.SemaphoreType.DMA,
    ],
)


def _phase2_body(counts_h, eg_h, xmin_h, ymin_h, xmax_h, ymax_h, lim_h,
                 oi_h, oj_h, opx_h, opy_h,
                 cnt_v, lim_v, obuf, gidx, gbg,
                 xmin_v, ymin_v, xmax_v, ymax_v,
                 soi, soj, sox, soy, sem):
    w = _wid()
    iota = lax.iota(jnp.int32, 16)

    cdmas = [
        pltpu.async_copy(xmin_h, xmin_v, sem),
        pltpu.async_copy(ymin_h, ymin_v, sem),
        pltpu.async_copy(xmax_h, xmax_v, sem),
        pltpu.async_copy(ymax_h, ymax_v, sem),
    ]
    pltpu.sync_copy(counts_h, cnt_v)
    pltpu.sync_copy(lim_h, lim_v)

    c0 = plsc.load_gather(cnt_v, [iota * 16])
    c1 = plsc.load_gather(cnt_v, [256 + iota * 16])
    lim = _vscalar(lim_v[...], iota)
    s0 = jnp.sum(c0)
    e0 = plsc.cumsum(c0) - c0
    e1 = plsc.cumsum(c1) - c1 + s0
    k_total = s0 + jnp.sum(c1)
    mink = jnp.minimum(k_total, lim)
    obuf[pl.ds(0, 16)] = e0
    obuf[pl.ds(16, 16)] = e1

    rs = w * SLOT

    def sixteen(t, _):
        g = rs + t * 16 + iota

        def owner(s, acc):
            o_s = plsc.load_gather(obuf, [jnp.full((16,), s, jnp.int32)])
            return acc + (g >= o_s).astype(jnp.int32)

        v = lax.fori_loop(0, NW, owner, jnp.zeros((16,), jnp.int32)) - 1
        o_v = plsc.load_gather(obuf, [v])
        addr = v * CAP + g - o_v
        addr = jnp.where(g < mink, addr, NW * CAP + iota)
        gidx[pl.ds(t * 16, 16)] = addr
        return 0

    lax.fori_loop(0, GROWS * 8, sixteen, 0)

    dmas = []
    for r in range(GROWS):
        sl = pl.ds(r * 128, 128)
        dmas.append(pltpu.async_copy(eg_h.at[gidx.at[sl]], gbg.at[sl], sem))
    for d in dmas + cdmas:
        d.wait()

    def decode(t, _):
        sl = pl.ds(t * 16, 16)
        g = gbg[sl]
        iv = g >> 13
        jv = g & 8191
        neg = g < 0
        jv = jnp.where(neg, -1, jv)
        gi = jnp.maximum(iv, 0)
        gj = jnp.maximum(jv, 0)
        xi0 = plsc.load_gather(xmin_v, [gi])
        yi0 = plsc.load_gather(ymin_v, [gi])
        xi1 = plsc.load_gather(xmax_v, [gi])
        yi1 = plsc.load_gather(ymax_v, [gi])
        xj0 = plsc.load_gather(xmin_v, [gj])
        yj0 = plsc.load_gather(ymin_v, [gj])
        xj1 = plsc.load_gather(xmax_v, [gj])
        yj1 = plsc.load_gather(ymax_v, [gj])
        ox = jnp.minimum(xi1, xj1) - jnp.maximum(xi0, xj0)
        oy = jnp.minimum(yi1, yj1) - jnp.maximum(yi0, yj0)
        soi[sl] = iv
        soj[sl] = jv
        sox[sl] = jnp.where(neg, 0.0, ox)
        soy[sl] = jnp.where(neg, 0.0, oy)
        return 0

    lax.fori_loop(0, GROWS * 8, decode, 0)

    dmas = [
        pltpu.async_copy(soi.at[pl.ds(0, SLOT)], oi_h.at[pl.ds(rs, SLOT)], sem),
        pltpu.async_copy(soj.at[pl.ds(0, SLOT)], oj_h.at[pl.ds(rs, SLOT)], sem),
        pltpu.async_copy(sox.at[pl.ds(0, SLOT)], opx_h.at[pl.ds(rs, SLOT)], sem),
        pltpu.async_copy(soy.at[pl.ds(0, SLOT)], opy_h.at[pl.ds(rs, SLOT)], sem),
    ]
    for d in dmas:
        d.wait()

    @pl.when(w == NW - 1)
    def _():
        src = pl.ds(SLOT, LIMIT - NW * SLOT)
        dst = pl.ds(rs + SLOT, LIMIT - NW * SLOT)
        d1 = pltpu.async_copy(soi.at[src], oi_h.at[dst], sem)
        d2 = pltpu.async_copy(soj.at[src], oj_h.at[dst], sem)
        d3 = pltpu.async_copy(sox.at[src], opx_h.at[dst], sem)
        d4 = pltpu.async_copy(soy.at[src], opy_h.at[dst], sem)
        d1.wait()
        d2.wait()
        d3.wait()
        d4.wait()


_PHASE2_KWARGS = dict(
    out_type=[
        jax.ShapeDtypeStruct((LIMIT,), jnp.int32),
        jax.ShapeDtypeStruct((LIMIT,), jnp.int32),
        jax.ShapeDtypeStruct((LIMIT,), jnp.float32),
        jax.ShapeDtypeStruct((LIMIT,), jnp.float32),
    ],
    scratch_types=[
        pltpu.VMEM((NW * 16,), jnp.int32),
        pltpu.VMEM((16,), jnp.int32),
        pltpu.VMEM((NW,), jnp.int32),
        pltpu.VMEM((GROWS * 128,), jnp.int32),
        pltpu.VMEM((GROWS * 128,), jnp.int32),
        pltpu.VMEM((NPAD,), jnp.float32),
        pltpu.VMEM((NPAD,), jnp.float32),
        pltpu.VMEM((NPAD,), jnp.float32),
        pltpu.VMEM((NPAD,), jnp.float32),
        pltpu.VMEM((GROWS * 128,), jnp.int32),
        pltpu.VMEM((GROWS * 128,), jnp.int32),
        pltpu.VMEM((GROWS * 128,), jnp.float32),
        pltpu.VMEM((GROWS * 128,), jnp.float32),
        pltpu.SemaphoreType.DMA,
    ],
)


@functools.cache
def _build_phases():
    mesh = plsc.VectorSubcoreMesh(core_axis_name="c", subcore_axis_name="s",
                                  num_cores=NC, num_subcores=NW // NC)
    params = pltpu.CompilerParams(needs_layout_passes=False,
                                  disable_bounds_checks=True)
    phase1 = pl.kernel(_phase1_body, mesh=mesh, compiler_params=params,
                       **_PHASE1_KWARGS)
    phase2 = pl.kernel(_phase2_body, mesh=mesh, compiler_params=params,
                       **_PHASE2_KWARGS)
    return phase1, phase2


def kernel(boxes, limit):
    boxes = boxes.astype(jnp.float32)
    padlo = jnp.full((NPAD - N,), 2e9, jnp.float32)
    padhi = jnp.full((NPAD - N,), -2e9, jnp.float32)
    xmin = jnp.concatenate([boxes[:, 0], padlo])
    ymin = jnp.concatenate([boxes[:, 1], padlo])
    xmax = jnp.concatenate([boxes[:, 2], padhi])
    ymax = jnp.concatenate([boxes[:, 3], padhi])
    bounds = jnp.asarray(_BOUNDS + [N] * (48 - len(_BOUNDS)), jnp.int32)
    lim = jnp.minimum(jnp.asarray(limit, jnp.int32), LIMIT)
    lim16 = jnp.full((16,), lim, jnp.int32)

    phase1, phase2 = _build_phases()
    counts, eg = phase1(xmin, ymin, xmax, ymax, bounds)
    oi, oj, opx, opy = phase2(counts, eg, xmin, ymin, xmax, ymax, lim16)

    pairs = jnp.stack([oi, oj], axis=-1)
    pen = jnp.stack([opx, opy], axis=-1)
    return pairs, pen


# speedup vs baseline: 3.6493x; 1.1440x over previous
import functools
import math

import jax
import jax.numpy as jnp
from jax import lax
from jax.experimental import pallas as pl
from jax.experimental.pallas import tpu as pltpu
from jax.experimental.pallas import tpu_sc as plsc

N = 5000
NPAD = 5120
LIMIT = 20000
OUT_PAD = LIMIT + 16
NW = 32
NC = 2
NB = 128
CAP = 20480
CHUNK = 512
SLOT = 624
GROWS = 6
UNROLL = 8


def _row_bounds():
    bs = [0]
    for b in range(1, NB):
        r = round(N * (1.0 - math.sqrt(1.0 - b / NB)))
        bs.append(max(bs[-1], min(r, N)))
    bs.append(N)
    return bs


_BOUNDS = _row_bounds()


def _wid():
    return lax.axis_index("s") * NC + lax.axis_index("c")


def _vscalar(vec, iota):
    return jnp.sum(jnp.where(iota == 0, vec, 0))


def _phase1_body(xmin_h, ymin_h, xmax_h, ymax_h, bounds_h,
                 counts_h, eg_h,
                 xmin_v, ymin_v, xmax_v, ymax_v, bounds_v,
                 bg, stage, sem):
    w = _wid()
    iota = lax.iota(jnp.int32, 16)

    pltpu.sync_copy(xmin_h, xmin_v)
    pltpu.sync_copy(ymin_h, ymin_v)
    pltpu.sync_copy(xmax_h, xmax_v)
    pltpu.sync_copy(ymax_h, ymax_v)
    pltpu.sync_copy(bounds_h, bounds_v)

    def row_body(i, off_v):
        isplat = jnp.full((16,), i, jnp.int32)
        ishift = isplat << 13
        rxmin = plsc.load_gather(xmin_v, [isplat])
        rymin = plsc.load_gather(ymin_v, [isplat])
        rxmax = plsc.load_gather(xmax_v, [isplat])
        rymax = plsc.load_gather(ymax_v, [isplat])
        lo = ((i + 1) // (16 * UNROLL)) * UNROLL

        def chunk_body(jc, offs):
            base = jc * 16
            cxmin = xmin_v[pl.ds(base, 16)]
            cymin = ymin_v[pl.ds(base, 16)]
            cxmax = xmax_v[pl.ds(base, 16)]
            cymax = ymax_v[pl.ds(base, 16)]
            ox = jnp.minimum(rxmax, cxmax) - jnp.maximum(rxmin, cxmin)
            oy = jnp.minimum(rymax, cymax) - jnp.maximum(rymin, cymin)
            jv = base + iota
            m = (jnp.minimum(ox, oy) > 0.0) & (jv > i)
            pos = plsc.cumsum(m.astype(jnp.int32))
            offc = jnp.minimum(offs, CAP - 16)
            plsc.store_scatter(bg, [offc + pos - 1], ishift | jv, mask=m)
            return offs + plsc.all_reduce_population_count(m)

        return plsc.parallel_loop(lo, NPAD // 16, unroll=UNROLL,
                                  carry=off_v)(chunk_body)

    def block_body(q, _):
        b = q * NW + w
        bvec_lo = plsc.load_gather(bounds_v, [jnp.full((16,), b, jnp.int32)])
        bvec_hi = plsc.load_gather(bounds_v,
                                   [jnp.full((16,), b + 1, jnp.int32)])
        rs = _vscalar(bvec_lo, iota)
        re = _vscalar(bvec_hi, iota)

        off_v = lax.fori_loop(rs, re, row_body, jnp.zeros((16,), jnp.int32))
        count = jnp.max(off_v)

        stage[...] = jnp.full((16,), count, jnp.int32)
        pltpu.sync_copy(stage, counts_h.at[pl.ds(b * 16, 16)])

        nch = (jnp.minimum(count, CAP) + CHUNK - 1) // CHUNK

        def spill(k, _):
            pltpu.sync_copy(bg.at[pl.ds(k * CHUNK, CHUNK)],
                            eg_h.at[pl.ds(b * CAP + k * CHUNK, CHUNK)])
            return 0

        lax.fori_loop(0, nch, spill, 0)
        return 0

    lax.fori_loop(0, NB // NW, block_body, 0)

    @pl.when(w == 0)
    def _():
        stage[...] = jnp.full((16,), -1, jnp.int32)
        pltpu.sync_copy(stage, eg_h.at[pl.ds(NB * CAP, 16)])


_PHASE1_KWARGS = dict(
    out_type=[
        jax.ShapeDtypeStruct((NB * 16,), jnp.int32),
        jax.ShapeDtypeStruct((NB * CAP + 16,), jnp.int32),
    ],
    scratch_types=[
        pltpu.VMEM((NPAD,), jnp.float32),
        pltpu.VMEM((NPAD,), jnp.float32),
        pltpu.VMEM((NPAD,), jnp.float32),
        pltpu.VMEM((NPAD,), jnp.float32),
        pltpu.VMEM((144,), jnp.int32),
        pltpu.VMEM((CAP,), jnp.int32),
        pltpu.VMEM((16,), jnp.int32),
        pltpu.SemaphoreType.DMA,
    ],
)


def _phase2_body(counts_h, eg_h, xmin_h, ymin_h, xmax_h, ymax_h, lim_h,
                 oi_h, oj_h, opx_h, opy_h,
                 cnt_v, lim_v, obuf, gidx, gbg,
                 xmin_v, ymin_v, xmax_v, ymax_v,
                 soi, soj, sox, soy, sem):
    w = _wid()
    iota = lax.iota(jnp.int32, 16)

    cdmas = [
        pltpu.async_copy(xmin_h, xmin_v, sem),
        pltpu.async_copy(ymin_h, ymin_v, sem),
        pltpu.async_copy(xmax_h, xmax_v, sem),
        pltpu.async_copy(ymax_h, ymax_v, sem),
    ]
    pltpu.sync_copy(counts_h, cnt_v)
    pltpu.sync_copy(lim_h, lim_v)

    lim = _vscalar(lim_v[...], iota)
    running = jnp.int32(0)
    for k in range(NB // 16):
        c_k = plsc.load_gather(cnt_v, [k * 256 + iota * 16])
        obuf[pl.ds(k * 16, 16)] = plsc.cumsum(c_k) - c_k + running
        running = running + jnp.sum(c_k)
    k_total = running
    mink = jnp.minimum(k_total, lim)

    rs = w * SLOT

    def sixteen(t, _):
        g = rs + t * 16 + iota
        lo = jnp.zeros((16,), jnp.int32)
        hi = jnp.full((16,), NB, jnp.int32)
        for _s in range(7):
            mid = (lo + hi) >> 1
            om = plsc.load_gather(obuf, [mid])
            cond = g >= om
            lo = jnp.where(cond, mid, lo)
            hi = jnp.where(cond, hi, mid)
        o_v = plsc.load_gather(obuf, [lo])
        addr = lo * CAP + g - o_v
        addr = jnp.where(g < mink, addr, NB * CAP + iota)
        gidx[pl.ds(t * 16, 16)] = addr
        return 0

    lax.fori_loop(0, GROWS * 8, sixteen, 0)

    dmas = []
    for r in range(GROWS):
        sl = pl.ds(r * 128, 128)
        dmas.append(pltpu.async_copy(eg_h.at[gidx.at[sl]], gbg.at[sl], sem))
    for d in dmas + cdmas:
        d.wait()

    def decode(t, _):
        sl = pl.ds(t * 16, 16)
        g = gbg[sl]
        iv = g >> 13
        jv = g & 8191
        neg = g < 0
        jv = jnp.where(neg, -1, jv)
        gi = jnp.maximum(iv, 0)
        gj = jnp.maximum(jv, 0)
        xi0 = plsc.load_gather(xmin_v, [gi])
        yi0 = plsc.load_gather(ymin_v, [gi])
        xi1 = plsc.load_gather(xmax_v, [gi])
        yi1 = plsc.load_gather(ymax_v, [gi])
        xj0 = plsc.load_gather(xmin_v, [gj])
        yj0 = plsc.load_gather(ymin_v, [gj])
        xj1 = plsc.load_gather(xmax_v, [gj])
        yj1 = plsc.load_gather(ymax_v, [gj])
        ox = jnp.minimum(xi1, xj1) - jnp.maximum(xi0, xj0)
        oy = jnp.minimum(yi1, yj1) - jnp.maximum(yi0, yj0)
        soi[sl] = iv
        soj[sl] = jv
        sox[sl] = jnp.where(neg, 0.0, ox)
        soy[sl] = jnp.where(neg, 0.0, oy)
        return 0

    lax.fori_loop(0, GROWS * 8, decode, 0)

    dmas = [
        pltpu.async_copy(soi.at[pl.ds(0, SLOT)], oi_h.at[pl.ds(rs, SLOT)], sem),
        pltpu.async_copy(soj.at[pl.ds(0, SLOT)], oj_h.at[pl.ds(rs, SLOT)], sem),
        pltpu.async_copy(sox.at[pl.ds(0, SLOT)], opx_h.at[pl.ds(rs, SLOT)], sem),
        pltpu.async_copy(soy.at[pl.ds(0, SLOT)], opy_h.at[pl.ds(rs, SLOT)], sem),
    ]
    for d in dmas:
        d.wait()

    @pl.when(w == NW - 1)
    def _():
        src = pl.ds(SLOT, LIMIT - NW * SLOT)
        dst = pl.ds(rs + SLOT, LIMIT - NW * SLOT)
        d1 = pltpu.async_copy(soi.at[src], oi_h.at[dst], sem)
        d2 = pltpu.async_copy(soj.at[src], oj_h.at[dst], sem)
        d3 = pltpu.async_copy(sox.at[src], opx_h.at[dst], sem)
        d4 = pltpu.async_copy(soy.at[src], opy_h.at[dst], sem)
        d1.wait()
        d2.wait()
        d3.wait()
        d4.wait()


_PHASE2_KWARGS = dict(
    out_type=[
        jax.ShapeDtypeStruct((LIMIT,), jnp.int32),
        jax.ShapeDtypeStruct((LIMIT,), jnp.int32),
        jax.ShapeDtypeStruct((LIMIT,), jnp.float32),
        jax.ShapeDtypeStruct((LIMIT,), jnp.float32),
    ],
    scratch_types=[
        pltpu.VMEM((NB * 16,), jnp.int32),
        pltpu.VMEM((16,), jnp.int32),
        pltpu.VMEM((NB,), jnp.int32),
        pltpu.VMEM((GROWS * 128,), jnp.int32),
        pltpu.VMEM((GROWS * 128,), jnp.int32),
        pltpu.VMEM((NPAD,), jnp.float32),
        pltpu.VMEM((NPAD,), jnp.float32),
        pltpu.VMEM((NPAD,), jnp.float32),
        pltpu.VMEM((NPAD,), jnp.float32),
        pltpu.VMEM((GROWS * 128,), jnp.int32),
        pltpu.VMEM((GROWS * 128,), jnp.int32),
        pltpu.VMEM((GROWS * 128,), jnp.float32),
        pltpu.VMEM((GROWS * 128,), jnp.float32),
        pltpu.SemaphoreType.DMA,
    ],
)


@functools.cache
def _build_phases():
    mesh = plsc.VectorSubcoreMesh(core_axis_name="c", subcore_axis_name="s",
                                  num_cores=NC, num_subcores=NW // NC)
    params = pltpu.CompilerParams(needs_layout_passes=False,
                                  disable_bounds_checks=True)
    phase1 = pl.kernel(_phase1_body, mesh=mesh, compiler_params=params,
                       **_PHASE1_KWARGS)
    phase2 = pl.kernel(_phase2_body, mesh=mesh, compiler_params=params,
                       **_PHASE2_KWARGS)
    return phase1, phase2


def kernel(boxes, limit):
    boxes = boxes.astype(jnp.float32)
    padlo = jnp.full((NPAD - N,), 2e9, jnp.float32)
    padhi = jnp.full((NPAD - N,), -2e9, jnp.float32)
    xmin = jnp.concatenate([boxes[:, 0], padlo])
    ymin = jnp.concatenate([boxes[:, 1], padlo])
    xmax = jnp.concatenate([boxes[:, 2], padhi])
    ymax = jnp.concatenate([boxes[:, 3], padhi])
    bounds = jnp.asarray(_BOUNDS + [N] * (144 - len(_BOUNDS)), jnp.int32)
    lim = jnp.minimum(jnp.asarray(limit, jnp.int32), LIMIT)
    lim16 = jnp.full((16,), lim, jnp.int32)

    phase1, phase2 = _build_phases()
    counts, eg = phase1(xmin, ymin, xmax, ymax, bounds)
    oi, oj, opx, opy = phase2(counts, eg, xmin, ymin, xmax, ymax, lim16)

    pairs = jnp.stack([oi, oj], axis=-1)
    pen = jnp.stack([opx, opy], axis=-1)
    return pairs, pen
